# Initial kernel scaffold; baseline (speedup 1.0000x reference)
import jax
import jax.numpy as jnp
from jax.experimental import pallas as pl


def kernel(x, edge_index, batch, Wl0, bl0, Wr0, Wl1, bl1, Wr1, Wl2, bl2, Wr2, Wl3, bl3, Wr3, Wh, bh):
    raise NotImplementedError("write your pallas kernel here")


# speedup vs baseline: 3.4154x; 3.4154x over previous
import functools

import jax
import jax.numpy as jnp
from jax import lax
from jax.experimental import pallas as pl
from jax.experimental.pallas import tpu as pltpu
from jax.experimental.pallas import tpu_sc as plsc

N = 10000
E = 320000
D = 128
G = 64

NC = 2
NS = 16
NW = NC * NS
CH = 128
EPAD = 323584
EPW = EPAD // NW
NCHUNK = EPW // CH
NROWS = 10240
RPW = NROWS // NS


def _make_sc_agg(width: int, count_only: bool = False):
    mesh = plsc.VectorSubcoreMesh(core_axis_name="c", subcore_axis_name="s")

    @functools.partial(
        pl.kernel,
        out_type=jax.ShapeDtypeStruct((NC, NROWS, width), jnp.float32),
        mesh=mesh,
        scratch_types=[
            pltpu.MemorySpace.VMEM_SHARED((NROWS, width), jnp.float32),
            pltpu.VMEM((CH,), jnp.int32),
            pltpu.VMEM((CH,), jnp.int32),
            pltpu.VMEM((CH, width), jnp.float32),
            pltpu.SemaphoreType.DMA,
        ],
    )
    def sc_agg(tab_hbm, src_hbm, dst_hbm, zro_hbm, out,
               acc_sp, sidx_v, didx_v, rows_v, sem):
        c = lax.axis_index("c")
        s = lax.axis_index("s")
        w = c * NS + s

        pltpu.sync_copy(zro_hbm, acc_sp.at[pl.ds(s * RPW, RPW)])
        if count_only:
            pltpu.sync_copy(tab_hbm, rows_v)
        plsc.subcore_barrier()

        def body(i, carry):
            base = w * EPW + i * CH
            pltpu.sync_copy(dst_hbm.at[pl.ds(base, CH)], didx_v)
            if not count_only:
                pltpu.sync_copy(src_hbm.at[pl.ds(base, CH)], sidx_v)
                pltpu.async_copy(tab_hbm.at[sidx_v], rows_v, sem).wait()
            pltpu.sync_copy(rows_v, acc_sp.at[didx_v], add=True)
            return carry

        lax.fori_loop(0, NCHUNK, body, 0)
        plsc.subcore_barrier()

        pltpu.sync_copy(acc_sp.at[pl.ds(s * RPW, RPW)],
                        out.at[c].at[pl.ds(s * RPW, RPW)])

    return sc_agg


def _layer_body(a0, a1, c0, c1, h, wlT, bl, wrT, o):
    cnt = c0[0, :, 0:1] + c1[0, :, 0:1]

    invd = 1.0 / jnp.maximum(cnt, 1.0)
    agg = (a0[0] + a1[0]) * invd
    y = (jnp.dot(agg, wlT[:], preferred_element_type=jnp.float32)
         + bl[:]
         + jnp.dot(h[:], wrT[:], preferred_element_type=jnp.float32))
    o[:] = jnp.where(y > 0, y, jnp.exp(y) - 1.0)


_BR = 1000


def _tc_layer(agg, cnt, h, wlT, bl, wrT):
    return pl.pallas_call(
        _layer_body,
        grid=(N // _BR,),
        in_specs=[
            pl.BlockSpec((1, _BR, D), lambda i: (0, i, 0)),
            pl.BlockSpec((1, _BR, D), lambda i: (1, i, 0)),
            pl.BlockSpec((1, _BR, D), lambda i: (0, i, 0)),
            pl.BlockSpec((1, _BR, D), lambda i: (1, i, 0)),
            pl.BlockSpec((_BR, D), lambda i: (i, 0)),
            pl.BlockSpec((D, D), lambda i: (0, 0)),
            pl.BlockSpec((1, D), lambda i: (0, 0)),
            pl.BlockSpec((D, D), lambda i: (0, 0)),
        ],
        out_specs=pl.BlockSpec((_BR, D), lambda i: (i, 0)),
        out_shape=jax.ShapeDtypeStruct((N, D), jnp.float32),
    )(agg, agg, cnt, cnt, h, wlT, bl, wrT)


def _head_body(bcol, h, whT, bh, o):
    gids = lax.broadcasted_iota(jnp.int32, (N, G), 1)
    oh = jnp.where(gids == bcol[:], 1.0, 0.0)
    g = lax.dot_general(oh, h[:], (((0,), (0,)), ((), ())),
                        preferred_element_type=jnp.float32)
    o[:] = jnp.dot(g, whT[:], preferred_element_type=jnp.float32) + bh[:]


def _tc_head(bcol, h, whT, bh):
    return pl.pallas_call(
        _head_body,
        out_shape=jax.ShapeDtypeStruct((G, 1), jnp.float32),
    )(bcol, h, whT, bh)


def kernel(x, edge_index, batch, Wl0, bl0, Wr0, Wl1, bl1, Wr1,
           Wl2, bl2, Wr2, Wl3, bl3, Wr3, Wh, bh):
    src = edge_index[0]
    dst = edge_index[1]
    pad = EPAD - E
    srcp = jnp.concatenate([src, jnp.zeros((pad,), jnp.int32)])
    dstp = jnp.concatenate([dst, jnp.full((pad,), N, jnp.int32)])

    sc128 = _make_sc_agg(D)
    sc_cnt = _make_sc_agg(D, count_only=True)

    ones_tab = jnp.ones((CH, D), jnp.float32)
    z128 = jnp.zeros((RPW, D), jnp.float32)
    cnt = sc_cnt(ones_tab, srcp, dstp, z128)

    params = [(Wl0, bl0, Wr0), (Wl1, bl1, Wr1), (Wl2, bl2, Wr2), (Wl3, bl3, Wr3)]
    h = x
    for Wl, bl, Wr in params:
        agg = sc128(h, srcp, dstp, z128)
        h = _tc_layer(agg, cnt, h, Wl.T, bl.reshape(1, D), Wr.T)

    bcol = batch.reshape(N, 1)
    return _tc_head(bcol, h, Wh.T, bh.reshape(1, 1))
